# Initial kernel scaffold; baseline (speedup 1.0000x reference)
import jax
import jax.numpy as jnp
from jax.experimental import pallas as pl


def kernel(x_cat, x_num, tables):
    raise NotImplementedError("write your pallas kernel here")


# speedup vs baseline: 3.7422x; 3.7422x over previous
import functools

import jax
import jax.numpy as jnp
from jax import lax
from jax.experimental import pallas as pl
from jax.experimental.pallas import tpu as pltpu
from jax.experimental.pallas import tpu_sc as plsc

_N_FIELDS = 26
_VOCAB = 1000
_EMB = 128
_BATCH = 4096
_N_NUM = 13
_OUT_W = _N_FIELDS * _EMB + _N_NUM

_NW = 32
_ROWS = _BATCH // _NW
_LANES = 16


def _body(xt_hbm, xnum_hbm, table_hbm, out_hbm, idx_v, rows_v, xn_v, sem):
    wid = lax.axis_index("s") * 2 + lax.axis_index("c")
    b0 = wid * _ROWS
    for f in range(_N_FIELDS):
        pltpu.sync_copy(xt_hbm.at[pl.ds(f * _BATCH + b0, _ROWS)], idx_v)
        if f:
            off = f * _VOCAB
            for j in range(_ROWS // _LANES):
                sl = pl.ds(j * _LANES, _LANES)
                idx_v[sl] = idx_v[sl] + off
        pltpu.async_copy(table_hbm.at[idx_v], rows_v, sem).wait()
        pltpu.sync_copy(
            rows_v, out_hbm.at[pl.ds(b0, _ROWS), pl.ds(f * _EMB, _EMB)]
        )
    pltpu.sync_copy(xnum_hbm.at[pl.ds(b0, _ROWS)], xn_v)
    pltpu.sync_copy(
        xn_v, out_hbm.at[pl.ds(b0, _ROWS), pl.ds(_N_FIELDS * _EMB, _N_NUM)]
    )


@jax.jit
def kernel(x_cat, x_num, tables):
    xt = x_cat.astype(jnp.int32).T.reshape(-1)
    table = tables.reshape(_N_FIELDS * _VOCAB, _EMB)
    run = functools.partial(
        pl.kernel,
        out_type=jax.ShapeDtypeStruct((_BATCH, _OUT_W), jnp.float32),
        mesh=plsc.VectorSubcoreMesh(core_axis_name="c", subcore_axis_name="s"),
        scratch_types=[
            pltpu.VMEM((_ROWS,), jnp.int32),
            pltpu.VMEM((_ROWS, _EMB), jnp.float32),
            pltpu.VMEM((_ROWS, _N_NUM), jnp.float32),
            pltpu.SemaphoreType.DMA,
        ],
    )(_body)
    return run(xt, x_num, table)


# speedup vs baseline: 4.9656x; 1.3269x over previous
import functools

import jax
import jax.numpy as jnp
from jax import lax
from jax.experimental import pallas as pl
from jax.experimental.pallas import tpu as pltpu
from jax.experimental.pallas import tpu_sc as plsc

_N_FIELDS = 26
_VOCAB = 1000
_EMB = 128
_BATCH = 4096
_N_NUM = 13
_OUT_W = _N_FIELDS * _EMB + _N_NUM

_NW = 32
_ROWS = _BATCH // _NW
_LANES = 16
_NB = 5


def _body(xt_hbm, xnum_hbm, table_hbm, out_hbm, idx_v, rows_v, xn_v, gsem, wsem, nsem):
    wid = lax.axis_index("s") * 2 + lax.axis_index("c")
    b0 = wid * _ROWS
    pltpu.sync_copy(xnum_hbm.at[pl.ds(b0, _ROWS)], xn_v)
    ncopy = pltpu.make_async_copy(
        xn_v, out_hbm.at[pl.ds(b0, _ROWS), pl.ds(_N_FIELDS * _EMB, _N_NUM)], nsem
    )
    ncopy.start()
    pltpu.sync_copy(xt_hbm.at[:, pl.ds(b0, _ROWS)], idx_v)
    for f in range(1, _N_FIELDS):
        off = f * _VOCAB
        for j in range(_ROWS // _LANES):
            sl = pl.ds(j * _LANES, _LANES)
            idx_v[f, sl] = idx_v[f, sl] + off

    gd = [None] * _N_FIELDS
    wd = [None] * _N_FIELDS

    def gstart(f):
        s = f % _NB
        gd[f] = pltpu.make_async_copy(
            table_hbm.at[idx_v.at[f]], rows_v.at[s], gsem.at[s]
        )
        gd[f].start()

    def wstart(f):
        s = f % _NB
        wd[f] = pltpu.make_async_copy(
            rows_v.at[s],
            out_hbm.at[pl.ds(b0, _ROWS), pl.ds(f * _EMB, _EMB)],
            wsem.at[s],
        )
        wd[f].start()

    for f in range(_NB - 1):
        gstart(f)
    for f in range(_N_FIELDS):
        gd[f].wait()
        wstart(f)
        nf = f + _NB - 1
        if nf < _N_FIELDS:
            if nf - _NB >= 0:
                wd[nf - _NB].wait()
            gstart(nf)
    for f in range(_N_FIELDS - _NB, _N_FIELDS):
        if f >= 0:
            wd[f].wait()
    ncopy.wait()


@jax.jit
def kernel(x_cat, x_num, tables):
    xt = x_cat.astype(jnp.int32).T
    table = tables.reshape(_N_FIELDS * _VOCAB, _EMB)
    run = functools.partial(
        pl.kernel,
        out_type=jax.ShapeDtypeStruct((_BATCH, _OUT_W), jnp.float32),
        mesh=plsc.VectorSubcoreMesh(core_axis_name="c", subcore_axis_name="s"),
        scratch_types=[
            pltpu.VMEM((_N_FIELDS, _ROWS), jnp.int32),
            pltpu.VMEM((_NB, _ROWS, _EMB), jnp.float32),
            pltpu.VMEM((_ROWS, _N_NUM), jnp.float32),
            pltpu.SemaphoreType.DMA((_NB,)),
            pltpu.SemaphoreType.DMA((_NB,)),
            pltpu.SemaphoreType.DMA,
        ],
    )(_body)
    return run(xt, x_num, table)
